# baseline (speedup 1.0000x reference)
import jax
import jax.numpy as jnp
from jax import lax
from jax.experimental import pallas as pl
from jax.experimental.pallas import tpu as pltpu
from jax.experimental.pallas import tpu_sc as plsc

_N = 32768
_E = 64
_K = 8
_L = 16
_NW = 32
_PER_W = _N // _NW
_GROUPS = _PER_W // _L
_ABLK = _PER_W * _K
_ASTR = _K * _N


def _tournament(leaves):
    while len(leaves) > 1:
        nxt = []
        for p in range(0, len(leaves), 2):
            (va, ia), (vb, ib) = leaves[p], leaves[p + 1]
            c = va >= vb
            nxt.append((jnp.where(c, va, vb), jnp.where(c, ia, ib)))
        leaves = nxt
    return leaves[0]


def _sc_body(x_hbm, bias_hbm, idx_hbm, w_hbm, xbuf, oibuf, owbuf, biasv, sem):
    nc = plsc.get_sparse_core_info().num_cores
    wid = lax.axis_index("s") * nc + lax.axis_index("c")
    lane = jnp.arange(_L, dtype=jnp.int32)
    neg = jnp.full((_L,), -jnp.inf, dtype=jnp.float32)
    t0 = wid * _PER_W

    descs = [
        pltpu.async_copy(x_hbm.at[pl.ds(a * _ASTR + t0 * _K, _ABLK)],
                         xbuf.at[pl.ds(a * _ABLK, _ABLK)], sem)
        for a in range(_E // _K)
    ]
    pltpu.sync_copy(bias_hbm, biasv)
    bvals = []
    for s in range(_E // _L):
        bvec = biasv[pl.ds(s * _L, _L)]
        bvals.extend(bvec[j] for j in range(_L))
    for d in descs:
        d.wait()

    def one_group(g):
        bg = (g >> 3) * 1024 + (g & 7) * _L
        vals, idxs = [], []
        for r in range(_K):
            subroots = []
            for s in range(4):
                leaves = []
                for j in range(16):
                    e = s * 16 + j
                    off = (e >> 3) * _ABLK + (e & 7) * 128
                    if r == 0:
                        ref = xbuf.at[pl.ds(bg + off, _L)]
                        v = ref[...] + bvals[e]
                        ref[...] = v
                    else:
                        v = xbuf[pl.ds(bg + off, _L)]
                    leaves.append((v, jnp.full((_L,), e, dtype=jnp.int32)))
                subroots.append(_tournament(leaves))
            m, am = _tournament(subroots)
            vals.append(m)
            idxs.append(am)
            if r < _K - 1:
                pos = ((am >> 3) * _ABLK + (am & 7) * 128 + bg) + lane
                plsc.store_scatter(xbuf, [pos], neg)
        es = [jnp.exp(v - vals[0]) for v in vals]
        ssum = es[0]
        for t in es[1:]:
            ssum = ssum + t
        rinv = 1.0 / ssum
        obg = (g >> 3) * 1024 + (g & 7) * _L
        for r in range(_K):
            oibuf[pl.ds(obg + r * 128, _L)] = idxs[r]
            owbuf[pl.ds(obg + r * 128, _L)] = es[r] * rinv

    def group_body(h, carry):
        one_group(h * 2)
        one_group(h * 2 + 1)
        return carry

    lax.fori_loop(0, _GROUPS // 2, group_body, 0)
    pltpu.sync_copy(oibuf, idx_hbm.at[pl.ds(t0 * _K, _ABLK)])
    pltpu.sync_copy(owbuf, w_hbm.at[pl.ds(t0 * _K, _ABLK)])


@jax.jit
def kernel(gate_logits, bias):
    xf = (gate_logits.reshape(_N // 128, 128, _E // _K, _K)
          .transpose(2, 0, 3, 1)
          .reshape(_N * _E))
    mesh = plsc.VectorSubcoreMesh(core_axis_name="c", subcore_axis_name="s")
    run = pl.kernel(
        _sc_body,
        out_type=[
            jax.ShapeDtypeStruct((_N * _K,), jnp.int32),
            jax.ShapeDtypeStruct((_N * _K,), jnp.float32),
        ],
        mesh=mesh,
        compiler_params=pltpu.CompilerParams(needs_layout_passes=False),
        scratch_types=[
            pltpu.VMEM((_E * _PER_W,), jnp.float32),
            pltpu.VMEM((_ABLK,), jnp.int32),
            pltpu.VMEM((_ABLK,), jnp.float32),
            pltpu.VMEM((_E,), jnp.float32),
            pltpu.SemaphoreType.DMA,
        ],
    )
    idxf, wf = run(xf, bias)
    idx = idxf.reshape(_N // 128, _K, 128).transpose(0, 2, 1).reshape(_N, _K)
    w = wf.reshape(_N // 128, _K, 128).transpose(0, 2, 1).reshape(_N, _K)
    return idx, w


# speedup vs baseline: 2.4714x; 2.4714x over previous
import jax
import jax.numpy as jnp
from jax import lax
from jax.experimental import pallas as pl
from jax.experimental.pallas import tpu as pltpu
from jax.experimental.pallas import tpu_sc as plsc

_N = 32768
_E = 64
_K = 8
_L = 16
_NW = 32
_PER_W = _N // _NW
_GROUPS = _PER_W // _L
_ABLK = _PER_W * _K
_ASTR = _K * _N


def _tournament(leaves):
    while len(leaves) > 1:
        nxt = []
        for p in range(0, len(leaves), 2):
            (va, ia), (vb, ib) = leaves[p], leaves[p + 1]
            c = va >= vb
            nxt.append((jnp.where(c, va, vb), jnp.where(c, ia, ib)))
        leaves = nxt
    return leaves[0]


def _sc_body(x_hbm, bias_hbm, idx_hbm, w_hbm, xbuf, oibuf, owbuf, biasv, sem):
    nc = plsc.get_sparse_core_info().num_cores
    wid = lax.axis_index("s") * nc + lax.axis_index("c")
    lane = jnp.arange(_L, dtype=jnp.int32)
    neg = jnp.full((_L,), -jnp.inf, dtype=jnp.float32)
    t0 = wid * _PER_W

    descs = [
        pltpu.async_copy(x_hbm.at[pl.ds(a * _ASTR + t0 * _K, _ABLK)],
                         xbuf.at[pl.ds(a * _ABLK, _ABLK)], sem)
        for a in range(_E // _K)
    ]
    pltpu.sync_copy(bias_hbm, biasv)
    bvals = []
    for s in range(_E // _L):
        bvec = biasv[pl.ds(s * _L, _L)]
        bvals.extend(bvec[j] for j in range(_L))
    for d in descs:
        d.wait()

    def one_group(g):
        bg = (g >> 3) * 1024 + (g & 7) * _L
        subroots = []
        for s in range(4):
            leaves = []
            for j in range(16):
                e = s * 16 + j
                off = (e >> 3) * _ABLK + (e & 7) * 128
                ref = xbuf.at[pl.ds(bg + off, _L)]
                v = ref[...] + bvals[e]
                ref[...] = v
                leaves.append((v, jnp.full((_L,), e, dtype=jnp.int32)))
            subroots.append(_tournament(leaves))
        m, am = _tournament(subroots)
        vals, idxs = [m], [am]
        for r in range(1, _K):
            pos = ((am >> 3) * _ABLK + (am & 7) * 128 + bg) + lane
            plsc.store_scatter(xbuf, [pos], neg)
            base_e = am & ~15
            pbase = base_e * 1024 + bg + lane
            leaves = []
            for j in range(16):
                off = (j >> 3) * _ABLK + (j & 7) * 128
                leaves.append((plsc.load_gather(xbuf, [pbase + off]),
                               base_e + j))
            nv, ni = _tournament(leaves)
            sid = am >> 4
            subroots = [
                (jnp.where(sid == s, nv, sv), jnp.where(sid == s, ni, si))
                for s, (sv, si) in enumerate(subroots)
            ]
            m, am = _tournament(subroots)
            vals.append(m)
            idxs.append(am)
        es = [jnp.exp(v - vals[0]) for v in vals]
        ssum = es[0]
        for t in es[1:]:
            ssum = ssum + t
        rinv = 1.0 / ssum
        obg = (g >> 3) * 1024 + (g & 7) * _L
        for r in range(_K):
            oibuf[pl.ds(obg + r * 128, _L)] = idxs[r]
            owbuf[pl.ds(obg + r * 128, _L)] = es[r] * rinv

    def group_body(g, carry):
        one_group(g)
        return carry

    lax.fori_loop(0, _GROUPS, group_body, 0)
    pltpu.sync_copy(oibuf, idx_hbm.at[pl.ds(t0 * _K, _ABLK)])
    pltpu.sync_copy(owbuf, w_hbm.at[pl.ds(t0 * _K, _ABLK)])


@jax.jit
def kernel(gate_logits, bias):
    xf = (gate_logits.reshape(_N // 128, 128, _E // _K, _K)
          .transpose(2, 0, 3, 1)
          .reshape(_N * _E))
    mesh = plsc.VectorSubcoreMesh(core_axis_name="c", subcore_axis_name="s")
    run = pl.kernel(
        _sc_body,
        out_type=[
            jax.ShapeDtypeStruct((_N * _K,), jnp.int32),
            jax.ShapeDtypeStruct((_N * _K,), jnp.float32),
        ],
        mesh=mesh,
        compiler_params=pltpu.CompilerParams(needs_layout_passes=False),
        scratch_types=[
            pltpu.VMEM((_E * _PER_W,), jnp.float32),
            pltpu.VMEM((_ABLK,), jnp.int32),
            pltpu.VMEM((_ABLK,), jnp.float32),
            pltpu.VMEM((_E,), jnp.float32),
            pltpu.SemaphoreType.DMA,
        ],
    )
    idxf, wf = run(xf, bias)
    idx = idxf.reshape(_N // 128, _K, 128).transpose(0, 2, 1).reshape(_N, _K)
    w = wf.reshape(_N // 128, _K, 128).transpose(0, 2, 1).reshape(_N, _K)
    return idx, w
